# Initial kernel scaffold; baseline (speedup 1.0000x reference)
import functools

import jax
import jax.numpy as jnp
from jax import lax
from jax.experimental import pallas as pl
from jax.experimental.pallas import tpu as pltpu, tpu_sc as plsc

N = 10000
E = 320000
D_IN = 128
D_HID = 128
D_OUT = 16

NC = 2
NS = 16
CHUNK = 128
EC32 = 80
EC16 = 2 * EC32
TOT_CHUNKS = 32 * EC32
EPAD = TOT_CHUNKS * CHUNK
RPT = N // NS
ACC_ROWS = N + 8

_mesh = functools.partial(
    plsc.VectorSubcoreMesh, core_axis_name="c", subcore_axis_name="s")


def _deg_kernel():
    @functools.partial(
        pl.kernel,
        out_type=jax.ShapeDtypeStruct((NC * N, 16), jnp.float32),
        mesh=_mesh(),
        scratch_types=[
            pltpu.VMEM((EC32, CHUNK), jnp.int32),
            pltpu.VMEM((CHUNK, 16), jnp.float32),
            pltpu.VMEM((RPT, 16), jnp.float32),
            pltpu.VMEM_SHARED((ACC_ROWS, 16), jnp.float32),
        ],
    )
    def deg_k(col_hbm, ones_hbm, zero_hbm, out_hbm, colv, onesv, stage, acc):
        c = lax.axis_index("c")
        s = lax.axis_index("s")
        wid = c * NS + s
        pltpu.sync_copy(col_hbm.at[pl.ds(wid * EC32, EC32)], colv)
        pltpu.sync_copy(ones_hbm, onesv)
        pltpu.sync_copy(zero_hbm.at[pl.ds(s * RPT, RPT)], stage)
        pltpu.sync_copy(stage, acc.at[pl.ds(s * RPT, RPT)])
        plsc.subcore_barrier()

        def body(j, carry):
            pltpu.sync_copy(onesv, acc.at[colv.at[j]], add=True)
            return carry

        lax.fori_loop(0, EC32, body, 0)
        plsc.subcore_barrier()
        pltpu.sync_copy(acc.at[pl.ds(s * RPT, RPT)], stage)
        pltpu.sync_copy(stage, out_hbm.at[pl.ds(c * N + s * RPT, RPT)])

    return deg_k


def _agg_kernel(depth, ec):
    @functools.partial(
        pl.kernel,
        out_type=jax.ShapeDtypeStruct((NC * N, depth), jnp.float32),
        mesh=_mesh(),
        scratch_types=[
            pltpu.VMEM((ec, CHUNK), jnp.int32),
            pltpu.VMEM((ec, CHUNK), jnp.int32),
            pltpu.VMEM((CHUNK, depth), jnp.float32),
            pltpu.VMEM((CHUNK, depth), jnp.float32),
            pltpu.VMEM((RPT, depth), jnp.float32),
            pltpu.VMEM_SHARED((ACC_ROWS, depth), jnp.float32),
            pltpu.SemaphoreType.DMA,
            pltpu.SemaphoreType.DMA,
        ],
    )
    def agg_k(row_hbm, col_hbm, tbl_hbm, out_hbm,
              rowv, colv, buf0, buf1, stage, acc, sem0, sem1):
        c = lax.axis_index("c")
        s = lax.axis_index("s")
        row_off = (c * NS + s) * ec
        if ec == EC16:
            col_off = s * ec
        else:
            col_off = (c * NS + s) * ec
        pltpu.sync_copy(row_hbm.at[pl.ds(row_off, ec)], rowv)
        pltpu.sync_copy(col_hbm.at[pl.ds(col_off, ec)], colv)
        pltpu.sync_copy(tbl_hbm.at[pl.ds(c * N + s * RPT, RPT)], stage)
        pltpu.sync_copy(stage, acc.at[pl.ds(s * RPT, RPT)])
        plsc.subcore_barrier()

        bufs = (buf0, buf1)
        sems = (sem0, sem1)
        for b in range(2):
            pltpu.async_copy(tbl_hbm.at[rowv.at[b]], bufs[b], sems[b])

        def body(t, carry):
            g = t * 2
            for b in range(2):
                j = g + b
                pltpu.make_async_copy(
                    tbl_hbm.at[rowv.at[j]], bufs[b], sems[b]).wait()
                pltpu.sync_copy(bufs[b], acc.at[colv.at[j]], add=True)

                @pl.when(j + 2 < ec)
                def _():
                    pltpu.async_copy(
                        tbl_hbm.at[rowv.at[j + 2]], bufs[b], sems[b])
            return carry

        lax.fori_loop(0, ec // 2, body, 0)
        plsc.subcore_barrier()
        pltpu.sync_copy(acc.at[pl.ds(s * RPT, RPT)], stage)
        pltpu.sync_copy(stage, out_hbm.at[pl.ds(c * N + s * RPT, RPT)])

    return agg_k


_BLK = 1000


def _k1(x, W1, degp):
    def body(x_ref, w_ref, degp_ref, hcat_ref, dis_ref):
        deg = degp_ref[0, :, 0] + degp_ref[1, :, 0] + 1.0
        dis = lax.rsqrt(deg)
        h = lax.dot_general(x_ref[...], w_ref[...],
                            (((1,), (1,)), ((), ())),
                            precision=lax.Precision.HIGHEST)
        hs = h * dis[:, None]
        hcat_ref[0] = hs[:, :64]
        hcat_ref[1] = hs[:, 64:]
        dis_ref[...] = dis[:, None]

    return pl.pallas_call(
        body,
        grid=(N // _BLK,),
        in_specs=[
            pl.BlockSpec((_BLK, D_IN), lambda i: (i, 0)),
            pl.BlockSpec((D_HID, D_IN), lambda i: (0, 0)),
            pl.BlockSpec((2, _BLK, 16), lambda i: (0, i, 0)),
        ],
        out_specs=[
            pl.BlockSpec((2, _BLK, 64), lambda i: (0, i, 0)),
            pl.BlockSpec((_BLK, 1), lambda i: (i, 0)),
        ],
        out_shape=[
            jax.ShapeDtypeStruct((2, N, 64), jnp.float32),
            jax.ShapeDtypeStruct((N, 1), jnp.float32),
        ],
    )(x, W1, degp)


def _k2(agg, dis, b1, W2):
    def body(agg_ref, dis_ref, b1_ref, w2_ref, out_ref):
        dis = dis_ref[...]
        z = jnp.concatenate([agg_ref[0], agg_ref[1]], axis=1)
        z = jnp.maximum(z * dis + b1_ref[...], 0.0)
        y = lax.dot_general(z, w2_ref[...],
                            (((1,), (1,)), ((), ())),
                            precision=lax.Precision.HIGHEST)
        out_ref[0] = y * dis
        out_ref[1] = jnp.zeros_like(y)

    return pl.pallas_call(
        body,
        grid=(N // _BLK,),
        in_specs=[
            pl.BlockSpec((2, _BLK, 64), lambda i: (0, i, 0)),
            pl.BlockSpec((_BLK, 1), lambda i: (i, 0)),
            pl.BlockSpec((1, D_HID), lambda i: (0, 0)),
            pl.BlockSpec((D_OUT, D_HID), lambda i: (0, 0)),
        ],
        out_specs=pl.BlockSpec((2, _BLK, D_OUT), lambda i: (0, i, 0)),
        out_shape=jax.ShapeDtypeStruct((2, N, D_OUT), jnp.float32),
    )(agg, dis, b1, W2)


def _k3(p2, dis, b2):
    def body(p_ref, dis_ref, b2_ref, out_ref):
        out_ref[...] = (p_ref[0] + p_ref[1]) * dis_ref[...] + b2_ref[...]

    return pl.pallas_call(
        body,
        grid=(N // _BLK,),
        in_specs=[
            pl.BlockSpec((2, _BLK, D_OUT), lambda i: (0, i, 0)),
            pl.BlockSpec((_BLK, 1), lambda i: (i, 0)),
            pl.BlockSpec((1, D_OUT), lambda i: (0, 0)),
        ],
        out_specs=pl.BlockSpec((_BLK, D_OUT), lambda i: (i, 0)),
        out_shape=jax.ShapeDtypeStruct((N, D_OUT), jnp.float32),
    )(p2, dis, b2)


def kernel(x, edge_index, W1, b1, W2, b2):
    pad = EPAD - E
    row = edge_index[0].astype(jnp.int32)
    col = edge_index[1].astype(jnp.int32)
    rowp = jnp.concatenate([row, jnp.zeros((pad,), jnp.int32)])
    colp = jnp.concatenate([col, jnp.full((pad,), N, jnp.int32)])
    row2 = jnp.concatenate([rowp, rowp + N]).reshape(2 * TOT_CHUNKS, CHUNK)
    col2d = colp.reshape(TOT_CHUNKS, CHUNK)

    ones16 = jnp.ones((CHUNK, 16), jnp.float32)
    zeros16 = jnp.zeros((N, 16), jnp.float32)

    degp = _deg_kernel()(col2d, ones16, zeros16).reshape(2, N, 16)

    hcat, dis = _k1(x, W1, degp)
    hcat = hcat.reshape(2 * N, 64)

    agg = _agg_kernel(64, EC16)(row2, col2d, hcat).reshape(2, N, 64)

    h2init = _k2(agg, dis, b1.reshape(1, D_HID), W2).reshape(2 * N, D_OUT)

    p2 = _agg_kernel(D_OUT, EC32)(row2, col2d, h2init).reshape(2, N, D_OUT)

    return _k3(p2, dis, b2.reshape(1, D_OUT))


# speedup vs baseline: 17.0331x; 17.0331x over previous
import functools

import jax
import jax.numpy as jnp
from jax import lax
from jax.experimental import pallas as pl
from jax.experimental.pallas import tpu as pltpu, tpu_sc as plsc

N = 10000
E = 320000
D_IN = 128
D_HID = 128
D_OUT = 16

NC = 2
NS = 16
NPAD = 10240
RPT = NPAD // NS
CHUNK = 128
EC32 = 80
EC16 = 2 * EC32
TOT_CHUNKS = 32 * EC32
EPAD = TOT_CHUNKS * CHUNK

_mesh = functools.partial(
    plsc.VectorSubcoreMesh, core_axis_name="c", subcore_axis_name="s")

_SC_PARAMS = pltpu.CompilerParams(use_tc_tiling_on_sc=False)


def _deg_kernel():
    @functools.partial(
        pl.kernel,
        out_type=jax.ShapeDtypeStruct((NC * NPAD, 16), jnp.float32),
        mesh=_mesh(),
        compiler_params=_SC_PARAMS,
        scratch_types=[
            pltpu.VMEM((EC32, CHUNK), jnp.int32),
            pltpu.VMEM((CHUNK, 16), jnp.float32),
            pltpu.VMEM((RPT, 16), jnp.float32),
            pltpu.VMEM_SHARED((NPAD, 16), jnp.float32),
        ],
    )
    def deg_k(col_hbm, ones_hbm, zero_hbm, out_hbm, colv, onesv, stage, acc):
        c = lax.axis_index("c")
        s = lax.axis_index("s")
        wid = c * NS + s
        pltpu.sync_copy(col_hbm.at[pl.ds(wid * EC32, EC32)], colv)
        pltpu.sync_copy(ones_hbm, onesv)
        pltpu.sync_copy(zero_hbm.at[pl.ds(s * RPT, RPT)], stage)
        pltpu.sync_copy(stage, acc.at[pl.ds(s * RPT, RPT)])
        plsc.subcore_barrier()

        def body(j, carry):
            pltpu.sync_copy(onesv, acc.at[colv.at[j]], add=True)
            return carry

        lax.fori_loop(0, EC32, body, 0)
        plsc.subcore_barrier()
        pltpu.sync_copy(acc.at[pl.ds(s * RPT, RPT)], stage)
        pltpu.sync_copy(stage, out_hbm.at[pl.ds(c * NPAD + s * RPT, RPT)])

    return deg_k


def _agg_kernel(depth, ec):
    @functools.partial(
        pl.kernel,
        out_type=jax.ShapeDtypeStruct((NC * NPAD, depth), jnp.float32),
        mesh=_mesh(),
        compiler_params=_SC_PARAMS,
        scratch_types=[
            pltpu.VMEM((ec, CHUNK), jnp.int32),
            pltpu.VMEM((ec, CHUNK), jnp.int32),
            pltpu.VMEM((CHUNK, depth), jnp.float32),
            pltpu.VMEM((CHUNK, depth), jnp.float32),
            pltpu.VMEM((RPT, depth), jnp.float32),
            pltpu.VMEM_SHARED((NPAD, depth), jnp.float32),
            pltpu.SemaphoreType.DMA,
            pltpu.SemaphoreType.DMA,
        ],
    )
    def agg_k(row_hbm, col_hbm, tbl_hbm, out_hbm,
              rowv, colv, buf0, buf1, stage, acc, sem0, sem1):
        c = lax.axis_index("c")
        s = lax.axis_index("s")
        row_off = (c * NS + s) * ec
        if ec == EC16:
            col_off = s * ec
        else:
            col_off = (c * NS + s) * ec
        pltpu.sync_copy(row_hbm.at[pl.ds(row_off, ec)], rowv)
        pltpu.sync_copy(col_hbm.at[pl.ds(col_off, ec)], colv)
        pltpu.sync_copy(tbl_hbm.at[pl.ds(c * NPAD + s * RPT, RPT)], stage)
        pltpu.sync_copy(stage, acc.at[pl.ds(s * RPT, RPT)])
        plsc.subcore_barrier()

        bufs = (buf0, buf1)
        sems = (sem0, sem1)
        for b in range(2):
            pltpu.async_copy(tbl_hbm.at[rowv.at[b]], bufs[b], sems[b])

        def body(t, carry):
            g = t * 2
            for b in range(2):
                j = g + b
                pltpu.make_async_copy(
                    tbl_hbm.at[rowv.at[j]], bufs[b], sems[b]).wait()
                pltpu.sync_copy(bufs[b], acc.at[colv.at[j]], add=True)

                @pl.when(j + 2 < ec)
                def _():
                    pltpu.async_copy(
                        tbl_hbm.at[rowv.at[j + 2]], bufs[b], sems[b])
            return carry

        lax.fori_loop(0, ec // 2, body, 0)
        plsc.subcore_barrier()
        pltpu.sync_copy(acc.at[pl.ds(s * RPT, RPT)], stage)
        pltpu.sync_copy(stage, out_hbm.at[pl.ds(c * NPAD + s * RPT, RPT)])

    return agg_k


_BLK = 640


def _k1(x, W1, degp):
    def body(x_ref, w_ref, degp_ref, hcat_ref, dis_ref):
        deg = degp_ref[0, :, 0] + degp_ref[1, :, 0] + 1.0
        dis = lax.rsqrt(deg)
        h = lax.dot_general(x_ref[...], w_ref[...],
                            (((1,), (1,)), ((), ())),
                            precision=lax.Precision.HIGHEST)
        hs = h * dis[:, None]
        for q in range(4):
            hcat_ref[q] = hs[:, 32 * q:32 * (q + 1)]
        dis_ref[...] = dis[:, None]

    return pl.pallas_call(
        body,
        grid=(NPAD // _BLK,),
        in_specs=[
            pl.BlockSpec((_BLK, D_IN), lambda i: (i, 0)),
            pl.BlockSpec((D_HID, D_IN), lambda i: (0, 0)),
            pl.BlockSpec((2, _BLK, 16), lambda i: (0, i, 0)),
        ],
        out_specs=[
            pl.BlockSpec((4, _BLK, 32), lambda i: (0, i, 0)),
            pl.BlockSpec((_BLK, 1), lambda i: (i, 0)),
        ],
        out_shape=[
            jax.ShapeDtypeStruct((4, NPAD, 32), jnp.float32),
            jax.ShapeDtypeStruct((NPAD, 1), jnp.float32),
        ],
    )(x, W1, degp)


def _k2(aggA, aggB, dis, b1, W2):
    def body(a_ref, b_ref, dis_ref, b1_ref, w2_ref, out_ref):
        dis = dis_ref[...]
        z = jnp.concatenate([a_ref[0], a_ref[1], b_ref[0], b_ref[1]], axis=1)
        z = jnp.maximum(z * dis + b1_ref[...], 0.0)
        y = lax.dot_general(z, w2_ref[...],
                            (((1,), (1,)), ((), ())),
                            precision=lax.Precision.HIGHEST)
        out_ref[0] = y * dis
        out_ref[1] = jnp.zeros_like(y)

    return pl.pallas_call(
        body,
        grid=(NPAD // _BLK,),
        in_specs=[
            pl.BlockSpec((2, _BLK, 32), lambda i: (0, i, 0)),
            pl.BlockSpec((2, _BLK, 32), lambda i: (0, i, 0)),
            pl.BlockSpec((_BLK, 1), lambda i: (i, 0)),
            pl.BlockSpec((1, D_HID), lambda i: (0, 0)),
            pl.BlockSpec((D_OUT, D_HID), lambda i: (0, 0)),
        ],
        out_specs=pl.BlockSpec((2, _BLK, D_OUT), lambda i: (0, i, 0)),
        out_shape=jax.ShapeDtypeStruct((2, NPAD, D_OUT), jnp.float32),
    )(aggA, aggB, dis, b1, W2)


def _k3(p2, dis, b2):
    def body(p_ref, dis_ref, b2_ref, out_ref):
        out_ref[...] = (p_ref[0] + p_ref[1]) * dis_ref[...] + b2_ref[...]

    return pl.pallas_call(
        body,
        grid=(NPAD // _BLK,),
        in_specs=[
            pl.BlockSpec((2, _BLK, D_OUT), lambda i: (0, i, 0)),
            pl.BlockSpec((_BLK, 1), lambda i: (i, 0)),
            pl.BlockSpec((1, D_OUT), lambda i: (0, 0)),
        ],
        out_specs=pl.BlockSpec((_BLK, D_OUT), lambda i: (i, 0)),
        out_shape=jax.ShapeDtypeStruct((NPAD, D_OUT), jnp.float32),
    )(p2, dis, b2)


def kernel(x, edge_index, W1, b1, W2, b2):
    pad = EPAD - E
    row = edge_index[0].astype(jnp.int32)
    col = edge_index[1].astype(jnp.int32)
    rowp = jnp.concatenate([row, jnp.zeros((pad,), jnp.int32)])
    colp = jnp.concatenate([col, jnp.full((pad,), N, jnp.int32)])
    row2 = jnp.concatenate([rowp, rowp + NPAD]).reshape(2 * TOT_CHUNKS, CHUNK)
    col2d = colp.reshape(TOT_CHUNKS, CHUNK)

    xp = jnp.zeros((NPAD, D_IN), x.dtype).at[:N].set(x)
    ones16 = jnp.ones((CHUNK, 16), jnp.float32)
    zeros16 = jnp.zeros((NPAD, 16), jnp.float32)

    degp = _deg_kernel()(col2d, ones16, zeros16).reshape(2, NPAD, 16)

    hcat, dis = _k1(xp, W1, degp)
    hcat = hcat.reshape(2, 2 * NPAD, 32)

    agg1 = _agg_kernel(32, EC16)
    aggA = agg1(row2, col2d, hcat[0]).reshape(2, NPAD, 32)
    aggB = agg1(row2, col2d, hcat[1]).reshape(2, NPAD, 32)

    h2init = _k2(aggA, aggB, dis, b1.reshape(1, D_HID),
                 W2).reshape(2 * NPAD, D_OUT)

    p2 = _agg_kernel(D_OUT, EC32)(row2, col2d, h2init).reshape(2, NPAD, D_OUT)

    return _k3(p2, dis, b2.reshape(1, D_OUT))[:N]


# speedup vs baseline: 19.5557x; 1.1481x over previous
import functools

import jax
import jax.numpy as jnp
from jax import lax
from jax.experimental import pallas as pl
from jax.experimental.pallas import tpu as pltpu, tpu_sc as plsc

N = 10000
E = 320000
D_IN = 128
D_HID = 128
D_OUT = 16

NC = 2
NS = 16
NPAD = 10240
RPT = NPAD // NS
CHUNK = 128
EC32 = 80
EC16 = 2 * EC32
TOT_CHUNKS = 32 * EC32
EPAD = TOT_CHUNKS * CHUNK
G = 5

_mesh = functools.partial(
    plsc.VectorSubcoreMesh, core_axis_name="c", subcore_axis_name="s")

_SC_PARAMS = pltpu.CompilerParams(use_tc_tiling_on_sc=False)


def _deg_kernel():
    NG = EC32 // G

    @functools.partial(
        pl.kernel,
        out_type=jax.ShapeDtypeStruct((NC * NPAD, 16), jnp.float32),
        mesh=_mesh(),
        compiler_params=_SC_PARAMS,
        scratch_types=[
            pltpu.VMEM((EC32, CHUNK), jnp.int32),
            pltpu.VMEM((CHUNK, 16), jnp.float32),
            pltpu.VMEM((RPT, 16), jnp.float32),
            pltpu.VMEM_SHARED((NPAD, 16), jnp.float32),
            pltpu.SemaphoreType.DMA,
            pltpu.SemaphoreType.DMA,
        ],
    )
    def deg_k(col_hbm, ones_hbm, zero_hbm, out_hbm,
              colv, onesv, stage, acc, semA, semB):
        c = lax.axis_index("c")
        s = lax.axis_index("s")
        wid = c * NS + s
        pltpu.sync_copy(col_hbm.at[pl.ds(wid * EC32, EC32)], colv)
        pltpu.sync_copy(ones_hbm, onesv)
        pltpu.sync_copy(zero_hbm.at[pl.ds(s * RPT, RPT)], stage)
        pltpu.sync_copy(stage, acc.at[pl.ds(s * RPT, RPT)])
        plsc.subcore_barrier()

        def scat(j, sem):
            pltpu.async_copy(onesv, acc.at[colv.at[j]], sem, add=True)

        def scat_wait(j, sem):
            pltpu.make_async_copy(onesv, acc.at[colv.at[j]], sem).wait()

        def body(t, carry):
            g = t * 2
            for k in range(G):
                scat(g * G + k, semA)

            @pl.when(g > 0)
            def _():
                for k in range(G):
                    scat_wait((g - 1) * G + k, semB)
            for k in range(G):
                scat((g + 1) * G + k, semB)
            for k in range(G):
                scat_wait(g * G + k, semA)
            return carry

        lax.fori_loop(0, NG // 2, body, 0)
        for k in range(G):
            scat_wait((NG - 1) * G + k, semB)
        plsc.subcore_barrier()
        pltpu.sync_copy(acc.at[pl.ds(s * RPT, RPT)], stage)
        pltpu.sync_copy(stage, out_hbm.at[pl.ds(c * NPAD + s * RPT, RPT)])

    return deg_k


def _agg_kernel(depth, n_pass, ec):
    NG = ec // G
    nq = n_pass * NC

    @functools.partial(
        pl.kernel,
        out_type=jax.ShapeDtypeStruct((nq * NPAD, depth), jnp.float32),
        mesh=_mesh(),
        compiler_params=_SC_PARAMS,
        scratch_types=[
            pltpu.VMEM((ec, CHUNK), jnp.int32),
            pltpu.VMEM((ec, CHUNK), jnp.int32),
            pltpu.VMEM((2 * G, CHUNK, depth), jnp.float32),
            pltpu.VMEM((RPT, depth), jnp.float32),
            pltpu.VMEM_SHARED((NPAD, depth), jnp.float32),
            pltpu.SemaphoreType.DMA,
            pltpu.SemaphoreType.DMA,
            pltpu.SemaphoreType.DMA,
            pltpu.SemaphoreType.DMA,
        ],
    )
    def agg_k(row_hbm, col_hbm, tbl_hbm, out_hbm,
              rowv, colv, bufs, stage, acc, gA, gB, sA, sB):
        c = lax.axis_index("c")
        s = lax.axis_index("s")
        if n_pass == 1:
            col_off = (c * NS + s) * ec
        else:
            col_off = s * ec
        pltpu.sync_copy(col_hbm.at[pl.ds(col_off, ec)], colv)

        def gath(j, bank, k, sem):
            pltpu.async_copy(tbl_hbm.at[rowv.at[j]], bufs.at[bank * G + k],
                             sem)

        def gath_wait(j, bank, k, sem):
            pltpu.make_async_copy(tbl_hbm.at[rowv.at[j]],
                                  bufs.at[bank * G + k], sem).wait()

        def scat(j, bank, k, sem):
            pltpu.async_copy(bufs.at[bank * G + k], acc.at[colv.at[j]], sem,
                             add=True)

        def scat_wait(j, bank, k, sem):
            pltpu.make_async_copy(bufs.at[bank * G + k],
                                  acc.at[colv.at[j]], sem).wait()

        for p in range(n_pass):
            q = 2 * p + c if n_pass == 2 else c
            if n_pass == 1:
                row_off = (c * NS + s) * ec
            else:
                row_off = q * TOT_CHUNKS + s * ec
            pltpu.sync_copy(row_hbm.at[pl.ds(row_off, ec)], rowv)
            pltpu.sync_copy(tbl_hbm.at[pl.ds(q * NPAD + s * RPT, RPT)], stage)
            pltpu.sync_copy(stage, acc.at[pl.ds(s * RPT, RPT)])
            plsc.subcore_barrier()

            for k in range(G):
                gath(k, 0, k, gA)

            def body(t, carry):
                g = t * 2

                @pl.when(g > 0)
                def _():
                    for k in range(G):
                        scat_wait((g - 1) * G + k, 1, k, sB)
                for k in range(G):
                    gath((g + 1) * G + k, 1, k, gB)
                for k in range(G):
                    gath_wait(g * G + k, 0, k, gA)
                for k in range(G):
                    scat(g * G + k, 0, k, sA)
                for k in range(G):
                    scat_wait(g * G + k, 0, k, sA)

                @pl.when(g + 2 < NG)
                def _():
                    for k in range(G):
                        gath((g + 2) * G + k, 0, k, gA)
                for k in range(G):
                    gath_wait((g + 1) * G + k, 1, k, gB)
                for k in range(G):
                    scat((g + 1) * G + k, 1, k, sB)
                return carry

            lax.fori_loop(0, NG // 2, body, 0)
            for k in range(G):
                scat_wait((NG - 1) * G + k, 1, k, sB)
            plsc.subcore_barrier()
            pltpu.sync_copy(acc.at[pl.ds(s * RPT, RPT)], stage)
            pltpu.sync_copy(stage,
                            out_hbm.at[pl.ds(q * NPAD + s * RPT, RPT)])
            if n_pass > 1 and p + 1 < n_pass:
                plsc.subcore_barrier()

    return agg_k


_BLK = 640


def _k1(x, W1, degp):
    def body(x_ref, w_ref, degp_ref, hcat_ref, dis_ref):
        deg = degp_ref[0, :, 0] + degp_ref[1, :, 0] + 1.0
        dis = lax.rsqrt(deg)
        h = lax.dot_general(x_ref[...], w_ref[...],
                            (((1,), (1,)), ((), ())),
                            precision=lax.Precision.HIGHEST)
        hs = h * dis[:, None]
        for q in range(4):
            hcat_ref[q] = hs[:, 32 * q:32 * (q + 1)]
        dis_ref[...] = dis[:, None]

    return pl.pallas_call(
        body,
        grid=(NPAD // _BLK,),
        in_specs=[
            pl.BlockSpec((_BLK, D_IN), lambda i: (i, 0)),
            pl.BlockSpec((D_HID, D_IN), lambda i: (0, 0)),
            pl.BlockSpec((2, _BLK, 16), lambda i: (0, i, 0)),
        ],
        out_specs=[
            pl.BlockSpec((4, _BLK, 32), lambda i: (0, i, 0)),
            pl.BlockSpec((_BLK, 1), lambda i: (i, 0)),
        ],
        out_shape=[
            jax.ShapeDtypeStruct((4, NPAD, 32), jnp.float32),
            jax.ShapeDtypeStruct((NPAD, 1), jnp.float32),
        ],
    )(x, W1, degp)


def _k2(agg, dis, b1, W2):
    def body(a_ref, dis_ref, b1_ref, w2_ref, out_ref):
        dis = dis_ref[...]
        z = jnp.concatenate([a_ref[0], a_ref[1], a_ref[2], a_ref[3]], axis=1)
        z = jnp.maximum(z * dis + b1_ref[...], 0.0)
        y = lax.dot_general(z, w2_ref[...],
                            (((1,), (1,)), ((), ())),
                            precision=lax.Precision.HIGHEST)
        out_ref[0] = y * dis
        out_ref[1] = jnp.zeros_like(y)

    return pl.pallas_call(
        body,
        grid=(NPAD // _BLK,),
        in_specs=[
            pl.BlockSpec((4, _BLK, 32), lambda i: (0, i, 0)),
            pl.BlockSpec((_BLK, 1), lambda i: (i, 0)),
            pl.BlockSpec((1, D_HID), lambda i: (0, 0)),
            pl.BlockSpec((D_OUT, D_HID), lambda i: (0, 0)),
        ],
        out_specs=pl.BlockSpec((2, _BLK, D_OUT), lambda i: (0, i, 0)),
        out_shape=jax.ShapeDtypeStruct((2, NPAD, D_OUT), jnp.float32),
    )(agg, dis, b1, W2)


def _k3(p2, dis, b2):
    def body(p_ref, dis_ref, b2_ref, out_ref):
        out_ref[...] = (p_ref[0] + p_ref[1]) * dis_ref[...] + b2_ref[...]

    return pl.pallas_call(
        body,
        grid=(NPAD // _BLK,),
        in_specs=[
            pl.BlockSpec((2, _BLK, D_OUT), lambda i: (0, i, 0)),
            pl.BlockSpec((_BLK, 1), lambda i: (i, 0)),
            pl.BlockSpec((1, D_OUT), lambda i: (0, 0)),
        ],
        out_specs=pl.BlockSpec((_BLK, D_OUT), lambda i: (i, 0)),
        out_shape=jax.ShapeDtypeStruct((NPAD, D_OUT), jnp.float32),
    )(p2, dis, b2)


def kernel(x, edge_index, W1, b1, W2, b2):
    pad = EPAD - E
    row = edge_index[0].astype(jnp.int32)
    col = edge_index[1].astype(jnp.int32)
    rowp = jnp.concatenate([row, jnp.zeros((pad,), jnp.int32)])
    colp = jnp.concatenate([col, jnp.full((pad,), N, jnp.int32)])
    row4 = jnp.concatenate(
        [rowp, rowp + NPAD, rowp + 2 * NPAD, rowp + 3 * NPAD]
    ).reshape(4 * TOT_CHUNKS, CHUNK)
    col2d = colp.reshape(TOT_CHUNKS, CHUNK)

    ones16 = jnp.ones((CHUNK, 16), jnp.float32)
    zeros16 = jnp.zeros((NPAD, 16), jnp.float32)

    degp = _deg_kernel()(col2d, ones16, zeros16).reshape(2, NPAD, 16)

    hcat, dis = _k1(x_pad(x), W1, degp)
    hcat = hcat.reshape(4 * NPAD, 32)

    agg = _agg_kernel(32, 2, EC16)(row4, col2d, hcat).reshape(4, NPAD, 32)

    h2init = _k2(agg, dis, b1.reshape(1, D_HID),
                 W2).reshape(2 * NPAD, D_OUT)

    p2 = _agg_kernel(D_OUT, 1, EC32)(row4, col2d, h2init).reshape(
        2, NPAD, D_OUT)

    return _k3(p2, dis, b2.reshape(1, D_OUT))[:N]


def x_pad(x):
    return jnp.zeros((NPAD, D_IN), x.dtype).at[:N].set(x)


# speedup vs baseline: 20.1380x; 1.0298x over previous
import functools

import jax
import jax.numpy as jnp
from jax import lax
from jax.experimental import pallas as pl
from jax.experimental.pallas import tpu as pltpu, tpu_sc as plsc

N = 10000
E = 320000
D_IN = 128
D_HID = 128
D_OUT = 16

NC = 2
NS = 16
NPAD = 10240
RPT = NPAD // NS
CHUNK = 128
EC32 = 80
TOT_CHUNKS = 32 * EC32
EPAD = TOT_CHUNKS * CHUNK

_mesh = functools.partial(
    plsc.VectorSubcoreMesh, core_axis_name="c", subcore_axis_name="s")

_SC_PARAMS = pltpu.CompilerParams(use_tc_tiling_on_sc=False)


def _deg_kernel():
    G = 5
    NG = EC32 // G

    @functools.partial(
        pl.kernel,
        out_type=jax.ShapeDtypeStruct((NC * NPAD, 16), jnp.float32),
        mesh=_mesh(),
        compiler_params=_SC_PARAMS,
        scratch_types=[
            pltpu.VMEM((EC32, CHUNK), jnp.int32),
            pltpu.VMEM((CHUNK, 16), jnp.float32),
            pltpu.VMEM((RPT, 16), jnp.float32),
            pltpu.VMEM_SHARED((NPAD, 16), jnp.float32),
            pltpu.SemaphoreType.DMA,
            pltpu.SemaphoreType.DMA,
        ],
    )
    def deg_k(col_hbm, ones_hbm, zero_hbm, out_hbm,
              colv, onesv, stage, acc, semA, semB):
        c = lax.axis_index("c")
        s = lax.axis_index("s")
        wid = c * NS + s
        pltpu.sync_copy(col_hbm.at[pl.ds(wid * EC32, EC32)], colv)
        pltpu.sync_copy(ones_hbm, onesv)
        pltpu.sync_copy(zero_hbm.at[pl.ds(s * RPT, RPT)], stage)
        pltpu.sync_copy(stage, acc.at[pl.ds(s * RPT, RPT)])
        plsc.subcore_barrier()

        def scat(j, sem):
            pltpu.async_copy(onesv, acc.at[colv.at[j]], sem, add=True)

        def scat_wait(j, sem):
            pltpu.make_async_copy(onesv, acc.at[colv.at[j]], sem).wait()

        def body(t, carry):
            g = t * 2
            for k in range(G):
                scat(g * G + k, semA)

            @pl.when(g > 0)
            def _():
                for k in range(G):
                    scat_wait((g - 1) * G + k, semB)
            for k in range(G):
                scat((g + 1) * G + k, semB)
            for k in range(G):
                scat_wait(g * G + k, semA)
            return carry

        lax.fori_loop(0, NG // 2, body, 0)
        for k in range(G):
            scat_wait((NG - 1) * G + k, semB)
        plsc.subcore_barrier()
        pltpu.sync_copy(acc.at[pl.ds(s * RPT, RPT)], stage)
        pltpu.sync_copy(stage, out_hbm.at[pl.ds(c * NPAD + s * RPT, RPT)])

    return deg_k


def _agg_kernel(depth, G, STG, mode):
    ec = (2 * EC32) if mode == "feat" else EC32
    NG = ec // G
    HOPS = RPT // STG

    @functools.partial(
        pl.kernel,
        out_type=jax.ShapeDtypeStruct((NC * NPAD, depth), jnp.float32),
        mesh=_mesh(),
        compiler_params=_SC_PARAMS,
        scratch_types=[
            pltpu.VMEM((ec, CHUNK), jnp.int32),
            pltpu.VMEM((ec, CHUNK), jnp.int32),
            pltpu.VMEM((2 * G, CHUNK, depth), jnp.float32),
            pltpu.VMEM((STG, depth), jnp.float32),
            pltpu.VMEM_SHARED((NPAD, depth), jnp.float32),
            pltpu.SemaphoreType.DMA,
            pltpu.SemaphoreType.DMA,
            pltpu.SemaphoreType.DMA,
            pltpu.SemaphoreType.DMA,
        ],
    )
    def agg_k(row_hbm, col_hbm, tbl_hbm, out_hbm,
              rowv, colv, bufs, stage, acc, gA, gB, sA, sB):
        c = lax.axis_index("c")
        s = lax.axis_index("s")
        if mode == "feat":
            row_off = c * TOT_CHUNKS + s * ec
            col_off = s * ec
        else:
            row_off = col_off = (c * NS + s) * ec
        pltpu.sync_copy(row_hbm.at[pl.ds(row_off, ec)], rowv)
        pltpu.sync_copy(col_hbm.at[pl.ds(col_off, ec)], colv)
        for h in range(HOPS):
            off = s * RPT + h * STG
            pltpu.sync_copy(tbl_hbm.at[pl.ds(c * NPAD + off, STG)], stage)
            pltpu.sync_copy(stage, acc.at[pl.ds(off, STG)])
        plsc.subcore_barrier()

        def gath(j, bank, k, sem):
            pltpu.async_copy(tbl_hbm.at[rowv.at[j]], bufs.at[bank * G + k],
                             sem)

        def gath_wait(j, bank, k, sem):
            pltpu.make_async_copy(tbl_hbm.at[rowv.at[j]],
                                  bufs.at[bank * G + k], sem).wait()

        def scat(j, bank, k, sem):
            pltpu.async_copy(bufs.at[bank * G + k], acc.at[colv.at[j]], sem,
                             add=True)

        def scat_wait(j, bank, k, sem):
            pltpu.make_async_copy(bufs.at[bank * G + k],
                                  acc.at[colv.at[j]], sem).wait()

        for k in range(G):
            gath(k, 0, k, gA)

        def body(t, carry):
            g = t * 2

            @pl.when(g > 0)
            def _():
                for k in range(G):
                    scat_wait((g - 1) * G + k, 1, k, sB)
            for k in range(G):
                gath((g + 1) * G + k, 1, k, gB)
            for k in range(G):
                gath_wait(g * G + k, 0, k, gA)
            for k in range(G):
                scat(g * G + k, 0, k, sA)
            for k in range(G):
                scat_wait(g * G + k, 0, k, sA)

            @pl.when(g + 2 < NG)
            def _():
                for k in range(G):
                    gath((g + 2) * G + k, 0, k, gA)
            for k in range(G):
                gath_wait((g + 1) * G + k, 1, k, gB)
            for k in range(G):
                scat((g + 1) * G + k, 1, k, sB)
            return carry

        lax.fori_loop(0, NG // 2, body, 0)
        for k in range(G):
            scat_wait((NG - 1) * G + k, 1, k, sB)
        plsc.subcore_barrier()
        for h in range(HOPS):
            off = s * RPT + h * STG
            pltpu.sync_copy(acc.at[pl.ds(off, STG)], stage)
            pltpu.sync_copy(stage, out_hbm.at[pl.ds(c * NPAD + off, STG)])

    return agg_k


_BLK = 640


def _k1(x, W1, degp):
    def body(x_ref, w_ref, degp_ref, hcat_ref, dis_ref):
        deg = degp_ref[0, :, 0] + degp_ref[1, :, 0] + 1.0
        dis = lax.rsqrt(deg)
        h = lax.dot_general(x_ref[...], w_ref[...],
                            (((1,), (1,)), ((), ())),
                            precision=lax.Precision.HIGHEST)
        hs = h * dis[:, None]
        hcat_ref[0] = hs[:, :64]
        hcat_ref[1] = hs[:, 64:]
        dis_ref[...] = dis[:, None]

    return pl.pallas_call(
        body,
        grid=(NPAD // _BLK,),
        in_specs=[
            pl.BlockSpec((_BLK, D_IN), lambda i: (i, 0)),
            pl.BlockSpec((D_HID, D_IN), lambda i: (0, 0)),
            pl.BlockSpec((2, _BLK, 16), lambda i: (0, i, 0)),
        ],
        out_specs=[
            pl.BlockSpec((2, _BLK, 64), lambda i: (0, i, 0)),
            pl.BlockSpec((_BLK, 1), lambda i: (i, 0)),
        ],
        out_shape=[
            jax.ShapeDtypeStruct((2, NPAD, 64), jnp.float32),
            jax.ShapeDtypeStruct((NPAD, 1), jnp.float32),
        ],
    )(x, W1, degp)


def _k2(agg, dis, b1, W2):
    def body(a_ref, dis_ref, b1_ref, w2_ref, out_ref):
        dis = dis_ref[...]
        z = jnp.concatenate([a_ref[0], a_ref[1]], axis=1)
        z = jnp.maximum(z * dis + b1_ref[...], 0.0)
        y = lax.dot_general(z, w2_ref[...],
                            (((1,), (1,)), ((), ())),
                            precision=lax.Precision.HIGHEST)
        out_ref[0] = y * dis
        out_ref[1] = jnp.zeros_like(y)

    return pl.pallas_call(
        body,
        grid=(NPAD // _BLK,),
        in_specs=[
            pl.BlockSpec((2, _BLK, 64), lambda i: (0, i, 0)),
            pl.BlockSpec((_BLK, 1), lambda i: (i, 0)),
            pl.BlockSpec((1, D_HID), lambda i: (0, 0)),
            pl.BlockSpec((D_OUT, D_HID), lambda i: (0, 0)),
        ],
        out_specs=pl.BlockSpec((2, _BLK, D_OUT), lambda i: (0, i, 0)),
        out_shape=jax.ShapeDtypeStruct((2, NPAD, D_OUT), jnp.float32),
    )(agg, dis, b1, W2)


def _k3(p2, dis, b2):
    def body(p_ref, dis_ref, b2_ref, out_ref):
        out_ref[...] = (p_ref[0] + p_ref[1]) * dis_ref[...] + b2_ref[...]

    return pl.pallas_call(
        body,
        grid=(NPAD // _BLK,),
        in_specs=[
            pl.BlockSpec((2, _BLK, D_OUT), lambda i: (0, i, 0)),
            pl.BlockSpec((_BLK, 1), lambda i: (i, 0)),
            pl.BlockSpec((1, D_OUT), lambda i: (0, 0)),
        ],
        out_specs=pl.BlockSpec((_BLK, D_OUT), lambda i: (i, 0)),
        out_shape=jax.ShapeDtypeStruct((NPAD, D_OUT), jnp.float32),
    )(p2, dis, b2)


def kernel(x, edge_index, W1, b1, W2, b2):
    pad = EPAD - E
    row = edge_index[0].astype(jnp.int32)
    col = edge_index[1].astype(jnp.int32)
    rowp = jnp.concatenate([row, jnp.zeros((pad,), jnp.int32)])
    colp = jnp.concatenate([col, jnp.full((pad,), N, jnp.int32)])
    row2d = jnp.concatenate([rowp, rowp + NPAD]).reshape(2 * TOT_CHUNKS, CHUNK)
    col2d = colp.reshape(TOT_CHUNKS, CHUNK)

    ones16 = jnp.ones((CHUNK, 16), jnp.float32)
    zeros16 = jnp.zeros((NPAD, 16), jnp.float32)

    degp = _deg_kernel()(col2d, ones16, zeros16).reshape(2, NPAD, 16)

    xp = jnp.zeros((NPAD, D_IN), x.dtype).at[:N].set(x)
    hcat, dis = _k1(xp, W1, degp)
    hcat = hcat.reshape(2 * NPAD, 64)

    agg = _agg_kernel(64, 2, 160, "feat")(row2d, col2d, hcat).reshape(
        2, NPAD, 64)

    h2init = _k2(agg, dis, b1.reshape(1, D_HID),
                 W2).reshape(2 * NPAD, D_OUT)

    p2 = _agg_kernel(D_OUT, 5, RPT, "edge")(row2d, col2d, h2init).reshape(
        2, NPAD, D_OUT)

    return _k3(p2, dis, b2.reshape(1, D_OUT))[:N]
